# Initial kernel scaffold; baseline (speedup 1.0000x reference)
import jax
import jax.numpy as jnp
from jax import lax
from jax.experimental import pallas as pl
from jax.experimental.pallas import tpu as pltpu, tpu_sc as plsc

ITEM_NUM = 1000000
ATTR_NUM = 100000
RATING_NUM = 5
EMBED_DIM = 32
ATTR_FNUM = 2
MAX_HIST_LEN = 200
BATCH = 4096
FIELD_NUM = 7

NC = 2
NS = 16
NW = NC * NS
B_PER_W = BATCH // NW
LP = MAX_HIST_LEN + 8
APL = 2 * MAX_HIST_LEN + 16
INV_L = 1.0 / MAX_HIST_LEN


def _zeros():
    return jnp.zeros((16,), jnp.float32)


def _sc_body(hi_hbm, ha_hbm, hr_hbm, iid_hbm, aid_hbm,
             item_t, attr_t, rating_t, out_hbm,
             outbuf, rt_v, ii_v, av_v,
             item_idx0, item_idx1, attr_idx0, attr_idx1, rate_idx0, rate_idx1,
             item_rows0, item_rows1, attr_rows0, attr_rows1,
             sem_idx0, sem_idx1, sem_rows0, sem_rows1, sem_a):
    item_idx = (item_idx0, item_idx1)
    attr_idx = (attr_idx0, attr_idx1)
    rate_idx = (rate_idx0, rate_idx1)
    item_rows = (item_rows0, item_rows1)
    attr_rows = (attr_rows0, attr_rows1)
    sem_idx = (sem_idx0, sem_idx1)
    sem_rows = (sem_rows0, sem_rows1)

    wid = lax.axis_index("s") * NC + lax.axis_index("c")
    base = wid * B_PER_W

    pltpu.sync_copy(rating_t, rt_v)

    pltpu.sync_copy(iid_hbm.at[wid], ii_v)
    pltpu.sync_copy(aid_hbm.at[wid], av_v)
    pltpu.async_copy(item_t.at[ii_v], item_rows0.at[pl.ds(0, 128)], sem_a)
    for c in range(2):
        pltpu.async_copy(attr_t.at[av_v.at[c]],
                         attr_rows0.at[pl.ds(c * 128, 128)], sem_a)
    pltpu.make_async_copy(item_t.at[pl.ds(0, 128)],
                          item_rows0.at[pl.ds(0, 128)], sem_a).wait()
    for c in range(2):
        pltpu.make_async_copy(attr_t.at[pl.ds(0, 128)],
                              attr_rows0.at[pl.ds(c * 128, 128)], sem_a).wait()

    @pl.loop(0, B_PER_W)
    def _copy_single(i):
        for v in range(2):
            sl = pl.ds(v * 16, 16)
            outbuf[i, 0, sl] = item_rows0[i, sl]
            outbuf[i, 1, sl] = attr_rows0[2 * i, sl]
            outbuf[i, 2, sl] = attr_rows0[2 * i + 1, sl]

    def start_idx(gb, slot):
        pltpu.async_copy(hi_hbm.at[gb], item_idx[slot], sem_idx[slot])
        pltpu.async_copy(ha_hbm.at[gb], attr_idx[slot], sem_idx[slot])
        pltpu.async_copy(hr_hbm.at[gb], rate_idx[slot], sem_idx[slot])

    def wait_idx(slot):
        pltpu.make_async_copy(hi_hbm.at[0], item_idx[slot], sem_idx[slot]).wait()
        pltpu.make_async_copy(ha_hbm.at[0], attr_idx[slot], sem_idx[slot]).wait()
        pltpu.make_async_copy(hr_hbm.at[0], rate_idx[slot], sem_idx[slot]).wait()

    def start_gathers(slot):
        for c in range(2):
            pltpu.async_copy(item_t.at[item_idx[slot].at[c]],
                             item_rows[slot].at[pl.ds(c * 104, 104)],
                             sem_rows[slot])
        for c in range(4):
            pltpu.async_copy(attr_t.at[attr_idx[slot].at[c]],
                             attr_rows[slot].at[pl.ds(c * 104, 104)],
                             sem_rows[slot])

    def wait_gathers(slot):
        for c in range(2):
            pltpu.make_async_copy(item_t.at[pl.ds(0, 104)],
                                  item_rows[slot].at[pl.ds(c * 104, 104)],
                                  sem_rows[slot]).wait()
        for c in range(4):
            pltpu.make_async_copy(attr_t.at[pl.ds(0, 104)],
                                  attr_rows[slot].at[pl.ds(c * 104, 104)],
                                  sem_rows[slot]).wait()

    def rating(k, slot):
        counts = [jnp.zeros((16,), jnp.int32) for _ in range(RATING_NUM)]
        for i in range(13):
            rv = rate_idx[slot][pl.ds(i * 16, 16)]
            for r in range(RATING_NUM):
                counts[r] += plsc.all_reduce_population_count(rv == r)
        acc = [_zeros(), _zeros()]
        for r in range(RATING_NUM):
            w = counts[r].astype(jnp.float32) * INV_L
            for v in range(2):
                acc[v] += w * rt_v[r, pl.ds(v * 16, 16)]
        for v in range(2):
            outbuf[k, 6, pl.ds(v * 16, 16)] = acc[v]

    def reduce(k, slot):
        ir = item_rows[slot]
        ar = attr_rows[slot]

        def body(l, accs):
            i0, i1, a00, a01, a10, a11 = accs
            s0, s1 = pl.ds(0, 16), pl.ds(16, 16)
            i0 = i0 + ir[l, s0]
            i1 = i1 + ir[l, s1]
            a00 = a00 + ar[2 * l, s0]
            a01 = a01 + ar[2 * l, s1]
            a10 = a10 + ar[2 * l + 1, s0]
            a11 = a11 + ar[2 * l + 1, s1]
            return i0, i1, a00, a01, a10, a11

        init = (_zeros(), _zeros(), _zeros(), _zeros(), _zeros(), _zeros())
        i0, i1, a00, a01, a10, a11 = lax.fori_loop(
            0, MAX_HIST_LEN, body, init, unroll=4)
        s0, s1 = pl.ds(0, 16), pl.ds(16, 16)
        outbuf[k, 3, s0] = i0 * INV_L
        outbuf[k, 3, s1] = i1 * INV_L
        outbuf[k, 4, s0] = a00 * INV_L
        outbuf[k, 4, s1] = a01 * INV_L
        outbuf[k, 5, s0] = a10 * INV_L
        outbuf[k, 5, s1] = a11 * INV_L

    def step(k, slot, do_idx, do_gather):
        wait_gathers(slot)
        rating(k, slot)
        if do_idx:
            start_idx(base + k + 2, slot)
        if do_gather:
            wait_idx(1 - slot)
            start_gathers(1 - slot)
        reduce(k, slot)

    start_idx(base + 0, 0)
    start_idx(base + 1, 1)
    wait_idx(0)
    start_gathers(0)

    @pl.loop(0, B_PER_W - 4, step=2)
    def _main(k):
        step(k, 0, True, True)
        step(k + 1, 1, True, True)

    step(B_PER_W - 4, 0, True, True)
    step(B_PER_W - 3, 1, True, True)
    step(B_PER_W - 2, 0, False, True)
    step(B_PER_W - 1, 1, False, False)

    pltpu.sync_copy(outbuf, out_hbm.at[pl.ds(base, B_PER_W)])


@jax.jit
def _run(hi_p, ha_p, hr_p, iid2, aid3, item_table, attr_table, rating_table):
    mesh = plsc.VectorSubcoreMesh(core_axis_name="c", subcore_axis_name="s")
    f = pl.kernel(
        _sc_body,
        out_type=jax.ShapeDtypeStruct((BATCH, FIELD_NUM, EMBED_DIM),
                                      jnp.float32),
        mesh=mesh,
        scratch_types=[
            pltpu.VMEM((B_PER_W, FIELD_NUM, EMBED_DIM), jnp.float32),
            pltpu.VMEM((RATING_NUM + 1, EMBED_DIM), jnp.float32),
            pltpu.VMEM((B_PER_W,), jnp.int32),
            pltpu.VMEM((2, 128), jnp.int32),
            pltpu.VMEM((2, 104), jnp.int32),
            pltpu.VMEM((2, 104), jnp.int32),
            pltpu.VMEM((4, 104), jnp.int32),
            pltpu.VMEM((4, 104), jnp.int32),
            pltpu.VMEM((LP,), jnp.int32),
            pltpu.VMEM((LP,), jnp.int32),
            pltpu.VMEM((LP, EMBED_DIM), jnp.float32),
            pltpu.VMEM((LP, EMBED_DIM), jnp.float32),
            pltpu.VMEM((APL, EMBED_DIM), jnp.float32),
            pltpu.VMEM((APL, EMBED_DIM), jnp.float32),
            pltpu.SemaphoreType.DMA,
            pltpu.SemaphoreType.DMA,
            pltpu.SemaphoreType.DMA,
            pltpu.SemaphoreType.DMA,
            pltpu.SemaphoreType.DMA,
        ],
    )
    return f(hi_p, ha_p, hr_p, iid2, aid3, item_table, attr_table,
             rating_table)


def kernel(hist_iid_seq, hist_aid_seq, hist_rate_seq, hist_seq_len, iid, aid,
           lb, item_table, attr_table, rating_table):
    del hist_seq_len, lb
    hi = hist_iid_seq.astype(jnp.int32)
    hi_p = jnp.pad(hi, ((0, 0), (0, LP - MAX_HIST_LEN))).reshape(BATCH, 2, 104)
    ha = hist_aid_seq.astype(jnp.int32).reshape(BATCH, 2 * MAX_HIST_LEN)
    ha_p = jnp.pad(ha, ((0, 0), (0, APL - 2 * MAX_HIST_LEN))).reshape(
        BATCH, 4, 104)
    hr_p = jnp.pad(hist_rate_seq.astype(jnp.int32),
                   ((0, 0), (0, LP - MAX_HIST_LEN)),
                   constant_values=RATING_NUM)
    iid2 = iid.astype(jnp.int32).reshape(NW, B_PER_W)
    aid3 = aid.astype(jnp.int32).reshape(NW, 2, B_PER_W)
    return _run(hi_p, ha_p, hr_p, iid2, aid3,
                item_table.astype(jnp.float32),
                attr_table.astype(jnp.float32),
                rating_table.astype(jnp.float32))


# speedup vs baseline: 7.4524x; 7.4524x over previous
import numpy as _np

import jax
import jax.numpy as jnp
from jax import lax
from jax.experimental import pallas as pl
from jax.experimental.pallas import tpu as pltpu, tpu_sc as plsc

ITEM_NUM = 1000000
ATTR_NUM = 100000
RATING_NUM = 5
EMBED_DIM = 32
ATTR_FNUM = 2
MAX_HIST_LEN = 200
BATCH = 4096
FIELD_NUM = 7

NC = 2
NS = 16
NW = NC * NS
B_PER_W = BATCH // NW
LP = MAX_HIST_LEN + 8
APL = 2 * MAX_HIST_LEN + 16
INV_L = 1.0 / MAX_HIST_LEN


def _zeros():
    return jnp.zeros((16,), jnp.float32)


def _sc_body(hi_hbm, ha_hbm, hr_hbm, iid_hbm, aid_hbm,
             item_t, attr_t, rating_t, out_hbm,
             outbuf, rt_v, ii_v, av_v,
             item_idx0, item_idx1, attr_idx0, attr_idx1, rate_idx0, rate_idx1,
             item_rows0, item_rows1, attr_rows0, attr_rows1,
             sem_idx0, sem_idx1, sem_rows0, sem_rows1, sem_a):
    item_idx = (item_idx0, item_idx1)
    attr_idx = (attr_idx0, attr_idx1)
    rate_idx = (rate_idx0, rate_idx1)
    item_rows = (item_rows0, item_rows1)
    attr_rows = (attr_rows0, attr_rows1)
    sem_idx = (sem_idx0, sem_idx1)
    sem_rows = (sem_rows0, sem_rows1)

    wid = lax.axis_index("s") * NC + lax.axis_index("c")
    base = wid * B_PER_W

    pltpu.sync_copy(rating_t, rt_v)

    pltpu.sync_copy(iid_hbm.at[wid], ii_v)
    pltpu.sync_copy(aid_hbm.at[wid], av_v)
    pltpu.async_copy(item_t.at[ii_v], item_rows0.at[pl.ds(0, 128)], sem_a)
    for c in range(2):
        pltpu.async_copy(attr_t.at[av_v.at[c]],
                         attr_rows0.at[pl.ds(c * 128, 128)], sem_a)
    pltpu.make_async_copy(item_t.at[pl.ds(0, 128)],
                          item_rows0.at[pl.ds(0, 128)], sem_a).wait()
    for c in range(2):
        pltpu.make_async_copy(attr_t.at[pl.ds(0, 128)],
                              attr_rows0.at[pl.ds(c * 128, 128)], sem_a).wait()

    @pl.loop(0, B_PER_W)
    def _copy_single(i):
        for v in range(2):
            sl = pl.ds(v * 16, 16)
            outbuf[i, 0, sl] = item_rows0[i, sl]
            outbuf[i, 1, sl] = attr_rows0[2 * i, sl]
            outbuf[i, 2, sl] = attr_rows0[2 * i + 1, sl]

    def start_idx(gb, slot):
        pltpu.async_copy(hi_hbm.at[gb], item_idx[slot], sem_idx[slot])
        pltpu.async_copy(ha_hbm.at[gb], attr_idx[slot], sem_idx[slot])
        pltpu.async_copy(hr_hbm.at[gb], rate_idx[slot], sem_idx[slot])

    def wait_idx(slot):
        pltpu.make_async_copy(hi_hbm.at[0], item_idx[slot], sem_idx[slot]).wait()
        pltpu.make_async_copy(ha_hbm.at[0], attr_idx[slot], sem_idx[slot]).wait()
        pltpu.make_async_copy(hr_hbm.at[0], rate_idx[slot], sem_idx[slot]).wait()

    def start_gathers(slot):
        for c in range(2):
            pltpu.async_copy(item_t.at[item_idx[slot].at[c]],
                             item_rows[slot].at[pl.ds(c * 104, 104)],
                             sem_rows[slot])
        for c in range(4):
            pltpu.async_copy(attr_t.at[attr_idx[slot].at[c]],
                             attr_rows[slot].at[pl.ds(c * 104, 104)],
                             sem_rows[slot])

    def wait_gathers(slot):
        for c in range(2):
            pltpu.make_async_copy(item_t.at[pl.ds(0, 104)],
                                  item_rows[slot].at[pl.ds(c * 104, 104)],
                                  sem_rows[slot]).wait()
        for c in range(4):
            pltpu.make_async_copy(attr_t.at[pl.ds(0, 104)],
                                  attr_rows[slot].at[pl.ds(c * 104, 104)],
                                  sem_rows[slot]).wait()

    def rating(k, slot):
        counts = [jnp.zeros((16,), jnp.int32) for _ in range(RATING_NUM)]
        one = jnp.ones((16,), jnp.int32)
        nil = jnp.zeros((16,), jnp.int32)
        lane = lax.broadcasted_iota(jnp.int32, (16,), 0)
        for i in range(13):
            rv = rate_idx[slot][pl.ds(i * 16, 16)]
            for r in range(RATING_NUM):
                counts[r] = counts[r] + jnp.where(rv == r, one, nil)
        acc = [_zeros(), _zeros()]
        for r in range(RATING_NUM):
            tot = counts[r]
            for sh in (8, 4, 2, 1):
                tot = tot + jnp.take_along_axis(tot, lane ^ sh, axis=0)
            w = tot.astype(jnp.float32) * INV_L
            for v in range(2):
                acc[v] += w * rt_v[r, pl.ds(v * 16, 16)]
        for v in range(2):
            outbuf[k, 6, pl.ds(v * 16, 16)] = acc[v]

    def reduce(k, slot):
        ir = item_rows[slot]
        ar = attr_rows[slot]

        def body(l, accs):
            i0, i1, a00, a01, a10, a11 = accs
            s0, s1 = pl.ds(0, 16), pl.ds(16, 16)
            i0 = i0 + ir[l, s0]
            i1 = i1 + ir[l, s1]
            a00 = a00 + ar[2 * l, s0]
            a01 = a01 + ar[2 * l, s1]
            a10 = a10 + ar[2 * l + 1, s0]
            a11 = a11 + ar[2 * l + 1, s1]
            return i0, i1, a00, a01, a10, a11

        init = (_zeros(), _zeros(), _zeros(), _zeros(), _zeros(), _zeros())
        i0, i1, a00, a01, a10, a11 = lax.fori_loop(
            0, MAX_HIST_LEN, body, init, unroll=4)
        s0, s1 = pl.ds(0, 16), pl.ds(16, 16)
        outbuf[k, 3, s0] = i0 * INV_L
        outbuf[k, 3, s1] = i1 * INV_L
        outbuf[k, 4, s0] = a00 * INV_L
        outbuf[k, 4, s1] = a01 * INV_L
        outbuf[k, 5, s0] = a10 * INV_L
        outbuf[k, 5, s1] = a11 * INV_L

    def step(k, slot, do_idx, do_gather):
        wait_gathers(slot)
        rating(k, slot)
        if do_idx:
            start_idx(base + k + 2, slot)
        if do_gather:
            wait_idx(1 - slot)
            start_gathers(1 - slot)
        reduce(k, slot)

    start_idx(base + 0, 0)
    start_idx(base + 1, 1)
    wait_idx(0)
    start_gathers(0)

    @pl.loop(0, B_PER_W - 4, step=2)
    def _main(k):
        step(k, 0, True, True)
        step(k + 1, 1, True, True)

    step(B_PER_W - 4, 0, True, True)
    step(B_PER_W - 3, 1, True, True)
    step(B_PER_W - 2, 0, False, True)
    step(B_PER_W - 1, 1, False, False)

    pltpu.sync_copy(outbuf, out_hbm.at[pl.ds(base, B_PER_W)])


@jax.jit
def _run(hi_p, ha_p, hr_p, iid2, aid3, item_table, attr_table, rating_table):
    mesh = plsc.VectorSubcoreMesh(core_axis_name="c", subcore_axis_name="s")
    f = pl.kernel(
        _sc_body,
        out_type=jax.ShapeDtypeStruct((BATCH, FIELD_NUM, EMBED_DIM),
                                      jnp.float32),
        mesh=mesh,
        scratch_types=[
            pltpu.VMEM((B_PER_W, FIELD_NUM, EMBED_DIM), jnp.float32),
            pltpu.VMEM((RATING_NUM + 1, EMBED_DIM), jnp.float32),
            pltpu.VMEM((B_PER_W,), jnp.int32),
            pltpu.VMEM((2, 128), jnp.int32),
            pltpu.VMEM((2, 104), jnp.int32),
            pltpu.VMEM((2, 104), jnp.int32),
            pltpu.VMEM((4, 104), jnp.int32),
            pltpu.VMEM((4, 104), jnp.int32),
            pltpu.VMEM((LP,), jnp.int32),
            pltpu.VMEM((LP,), jnp.int32),
            pltpu.VMEM((LP, EMBED_DIM), jnp.float32),
            pltpu.VMEM((LP, EMBED_DIM), jnp.float32),
            pltpu.VMEM((APL, EMBED_DIM), jnp.float32),
            pltpu.VMEM((APL, EMBED_DIM), jnp.float32),
            pltpu.SemaphoreType.DMA,
            pltpu.SemaphoreType.DMA,
            pltpu.SemaphoreType.DMA,
            pltpu.SemaphoreType.DMA,
            pltpu.SemaphoreType.DMA,
        ],
        compiler_params=pltpu.CompilerParams(use_tc_tiling_on_sc=False),
    )
    return f(hi_p, ha_p, hr_p, iid2, aid3, item_table, attr_table,
             rating_table)


def kernel(hist_iid_seq, hist_aid_seq, hist_rate_seq, hist_seq_len, iid, aid,
           lb, item_table, attr_table, rating_table):
    del hist_seq_len, lb
    hi = hist_iid_seq.astype(jnp.int32)
    hi_p = jnp.pad(hi, ((0, 0), (0, LP - MAX_HIST_LEN))).reshape(BATCH, 2, 104)
    ha = hist_aid_seq.astype(jnp.int32).reshape(BATCH, 2 * MAX_HIST_LEN)
    ha_p = jnp.pad(ha, ((0, 0), (0, APL - 2 * MAX_HIST_LEN))).reshape(
        BATCH, 4, 104)
    hr_p = jnp.pad(hist_rate_seq.astype(jnp.int32),
                   ((0, 0), (0, LP - MAX_HIST_LEN)),
                   constant_values=RATING_NUM)
    iid2 = iid.astype(jnp.int32).reshape(NW, B_PER_W)
    aid3 = aid.astype(jnp.int32).reshape(NW, 2, B_PER_W)
    return _run(hi_p, ha_p, hr_p, iid2, aid3,
                item_table.astype(jnp.float32),
                attr_table.astype(jnp.float32),
                rating_table.astype(jnp.float32))
